# baseline (speedup 1.0000x reference)
import jax
import jax.numpy as jnp
from jax import lax
from jax.experimental import pallas as pl
from jax.experimental.pallas import tpu as pltpu
from jax.experimental.pallas import tpu_sc as plsc

B = 16384
D = 32
L = 16

_info = plsc.get_sparse_core_info()
NC, NS = _info.num_cores, _info.num_subcores
NW = NC * NS
ROWS_PER_W = B // NW
CH = 64
N_CHUNKS = ROWS_PER_W // CH
QPC = CH // L
NBLK = B // CH
PACK = 128 // D


def _sc_body(eu, ei, eru, eri, ixo_hbm, diff_out, reg_out,
             b0, b1, b2, b3, b4, b5, r0, r1, r2, r3, r4, r5,
             ixo_v, diff_v, vec_v, sem):
    wid = lax.axis_index("s") * NC + lax.axis_index("c")
    bigs = [b0, b1, b2, b3, b4, b5]
    rels = [r0, r1, r2, r3, r4, r5]
    big_tbl = [eu, eu, eu, ei, ei, ei]
    rel_tbl = [eru, eru, eru, eri, eri, eri]
    iota = lax.iota(jnp.int32, L)

    def chunk_body(g, sq_acc):
        blk = wid * N_CHUNKS + g
        pltpu.sync_copy(ixo_hbm.at[blk], ixo_v)
        copies = []
        for t in range(6):
            copies.append(pltpu.async_copy(
                big_tbl[t].at[ixo_v.at[t]], bigs[t], sem))
        for t in range(6):
            copies.append(pltpu.async_copy(
                rel_tbl[t].at[ixo_v.at[6 + t]], rels[t], sem))
        for c in copies:
            c.wait()

        def group_body(qi, sq):
            rows = qi * L + iota
            offs = [ixo_v[12 + t, pl.ds(qi * L, L)] for t in range(6)]
            acc = jnp.zeros((L,), jnp.float32)
            for d in range(D):
                bv = [plsc.load_gather(bigs[t], [rows, offs[t] + d])
                      for t in range(6)]
                rv = [plsc.load_gather(rels[t],
                                       [rows, jnp.full((L,), d, jnp.int32)])
                      for t in range(6)]
                base = bv[0] + bv[3] + rv[0] + rv[3]
                pos = bv[1] + bv[4] + rv[1] + rv[4]
                neg = bv[2] + bv[5] + rv[2] + rv[5]
                acc = acc + base * (neg - pos)
                for v in bv:
                    sq = sq + v * v
                for v in rv:
                    sq = sq + v * v
            diff_v[pl.ds((g * QPC + qi) * L, L)] = acc
            return sq

        return lax.fori_loop(0, QPC, group_body, sq_acc)

    acc_sq = lax.fori_loop(0, N_CHUNKS, chunk_body,
                           jnp.zeros((L,), jnp.float32))
    vec_v[...] = acc_sq
    pltpu.sync_copy(diff_v, diff_out.at[pl.ds(wid * ROWS_PER_W, ROWS_PER_W)])
    pltpu.sync_copy(vec_v, reg_out.at[pl.ds(wid * L, L)])


def _finalize_body(diff_ref, reg_ref, loss_ref, regloss_ref):
    x = diff_ref[...]
    sp = jnp.maximum(x, 0.0) + jnp.log1p(jnp.exp(-jnp.abs(x)))
    loss_ref[0, 0] = jnp.sum(sp) / float(B)
    regloss_ref[0, 0] = 0.5 * jnp.sum(reg_ref[...]) / float(B)


def kernel(user, item, user_pos, item_pos, user_neg, item_neg,
           rel_u, pos_rel_u, neg_rel_u, rel_i, pos_rel_i, neg_rel_i,
           embed_user, embed_item, embed_rel_u, embed_rel_i):
    big = jnp.stack([user, user_pos, user_neg,
                     item, item_pos, item_neg]).astype(jnp.int32)
    rel = jnp.stack([rel_u, pos_rel_u, neg_rel_u,
                     rel_i, pos_rel_i, neg_rel_i]).astype(jnp.int32)
    ixo = jnp.concatenate([big // PACK, rel, (big % PACK) * D], axis=0)
    ixo = ixo.reshape(18, NBLK, CH).transpose(1, 0, 2)

    eu2 = embed_user.reshape(-1, PACK * D)
    ei2 = embed_item.reshape(-1, PACK * D)

    sc = pl.kernel(
        _sc_body,
        mesh=plsc.VectorSubcoreMesh(core_axis_name="c", subcore_axis_name="s"),
        compiler_params=pltpu.CompilerParams(use_tc_tiling_on_sc=False,
                                             needs_layout_passes=False),
        out_type=[jax.ShapeDtypeStruct((B,), jnp.float32),
                  jax.ShapeDtypeStruct((NW * L,), jnp.float32)],
        scratch_types=(
            [pltpu.VMEM((CH, PACK * D), jnp.float32) for _ in range(6)]
            + [pltpu.VMEM((CH, D), jnp.float32) for _ in range(6)]
            + [pltpu.VMEM((18, CH), jnp.int32),
               pltpu.VMEM((ROWS_PER_W,), jnp.float32),
               pltpu.VMEM((L,), jnp.float32),
               pltpu.SemaphoreType.DMA]),
    )
    diff, reg_part = sc(eu2, ei2, embed_rel_u, embed_rel_i, ixo)

    loss, reg_loss = pl.pallas_call(
        _finalize_body,
        out_shape=[jax.ShapeDtypeStruct((1, 1), jnp.float32),
                   jax.ShapeDtypeStruct((1, 1), jnp.float32)],
        out_specs=[pl.BlockSpec(memory_space=pltpu.SMEM),
                   pl.BlockSpec(memory_space=pltpu.SMEM)],
    )(diff.reshape(B // 128, 128), reg_part.reshape(NW * L // 128, 128))
    return (loss[0, 0], reg_loss[0, 0])
